# baseline (speedup 1.0000x reference)
import functools

import jax
import jax.numpy as jnp
from jax.experimental import pallas as pl
from jax.experimental.pallas import tpu as pltpu

_THRESH = 0.055
_N = 500
_E = 128


def _ccn2_body(feat_ref, featT_ref, w0t_ref, w0b_ref, w2t_ref, w2b_ref,
               out_ref, mean_ref):
    f = feat_ref[0]
    ft = featT_ref[0]
    xc = f[:, 0:1]
    yc = f[:, 1:2]
    xr = ft[0:1, :]
    yr = ft[1:2, :]
    dx = xc - xr
    dy = yc - yr
    dist2 = dx * dx + dy * dy
    Ab = (dist2 <= _THRESH * _THRESH).astype(jnp.bfloat16)

    fv0 = jnp.maximum(
        jnp.dot(f, w0t_ref[...], preferred_element_type=jnp.float32)
        + w0b_ref[...], 0.0)
    fh = fv0.astype(jnp.bfloat16)
    fl = (fv0 - fh.astype(jnp.float32)).astype(jnp.bfloat16)
    fv1 = (jnp.dot(Ab, fh, preferred_element_type=jnp.float32)
           + jnp.dot(Ab, fl, preferred_element_type=jnp.float32))

    Ai = (dist2 <= _THRESH * _THRESH).astype(jnp.int8)
    C = jnp.dot(Ai, Ai, preferred_element_type=jnp.int32)
    hop2 = C > 0
    B2i = hop2.astype(jnp.int8)
    D = jnp.dot(B2i, Ai, preferred_element_type=jnp.int32)
    M = jnp.where(hop2, D.astype(jnp.float32), 0.0)

    fv2 = jnp.dot(M, fv1, preferred_element_type=jnp.float32)
    Fv2 = jnp.maximum(
        jnp.dot(fv2, w2t_ref[...], preferred_element_type=jnp.float32)
        + w2b_ref[...], 0.0)
    out_ref[0] = Fv2
    mean_ref[0, 0] = jnp.mean(Fv2, axis=0)


@functools.partial(jax.jit, static_argnames=())
def kernel(loc, deadline, depot, W0_w, W0_b, W2_w, W2_b):
    B = loc.shape[0]
    locations = jnp.concatenate([depot[:, None, :], loc], axis=1)
    td = jnp.concatenate(
        [jnp.zeros((B, 1), deadline.dtype), deadline], axis=1)
    feat = jnp.concatenate([locations, td[..., None]], axis=-1)
    featT = jnp.swapaxes(feat, 1, 2)
    w0t = W0_w.T
    w2t = W2_w.T
    w0b = W0_b[None, :]
    w2b = W2_b[None, :]

    grid = (B,)
    out_shape = (
        jax.ShapeDtypeStruct((B, _N, _E), jnp.float32),
        jax.ShapeDtypeStruct((B, 1, _E), jnp.float32),
    )
    Fv2, mean = pl.pallas_call(
        _ccn2_body,
        grid=grid,
        in_specs=[
            pl.BlockSpec((1, _N, 3), lambda b: (b, 0, 0)),
            pl.BlockSpec((1, 3, _N), lambda b: (b, 0, 0)),
            pl.BlockSpec((3, _E), lambda b: (0, 0)),
            pl.BlockSpec((1, _E), lambda b: (0, 0)),
            pl.BlockSpec((_E, _E), lambda b: (0, 0)),
            pl.BlockSpec((1, _E), lambda b: (0, 0)),
        ],
        out_specs=(
            pl.BlockSpec((1, _N, _E), lambda b: (b, 0, 0)),
            pl.BlockSpec((1, 1, _E), lambda b: (b, 0, 0)),
        ),
        out_shape=out_shape,
        compiler_params=pltpu.CompilerParams(
            dimension_semantics=("arbitrary",),
        ),
    )(feat, featT, w0t, w0b, w2t, w2b)
    return Fv2, mean[:, 0, :]


# speedup vs baseline: 1.0191x; 1.0191x over previous
import functools

import jax
import jax.numpy as jnp
from jax.experimental import pallas as pl
from jax.experimental.pallas import tpu as pltpu

_THRESH = 0.055
_N = 500
_E = 128


def _ccn2_body(feat_ref, featT_ref, w0t_ref, w0b_ref, w2t_ref, w2b_ref,
               out_ref, mean_ref):
    f = feat_ref[0]
    ft = featT_ref[0]
    xc = f[:, 0:1]
    yc = f[:, 1:2]
    xr = ft[0:1, :]
    yr = ft[1:2, :]
    dx = xc - xr
    dy = yc - yr
    dist2 = dx * dx + dy * dy
    Ab = (dist2 <= _THRESH * _THRESH).astype(jnp.bfloat16)

    fv0 = jnp.maximum(
        jnp.dot(f, w0t_ref[...], preferred_element_type=jnp.float32)
        + w0b_ref[...], 0.0)
    fh = fv0.astype(jnp.bfloat16)
    fl = (fv0 - fh.astype(jnp.float32)).astype(jnp.bfloat16)
    fv1 = (jnp.dot(Ab, fh, preferred_element_type=jnp.float32)
           + jnp.dot(Ab, fl, preferred_element_type=jnp.float32))

    C = jnp.dot(Ab, Ab, preferred_element_type=jnp.float32)
    hop2 = C > 0
    B2b = hop2.astype(jnp.bfloat16)
    D = jnp.dot(B2b, Ab, preferred_element_type=jnp.float32)
    M = jnp.where(hop2, D, 0.0)

    fv2 = jnp.dot(M, fv1, preferred_element_type=jnp.float32)
    Fv2 = jnp.maximum(
        jnp.dot(fv2, w2t_ref[...], preferred_element_type=jnp.float32)
        + w2b_ref[...], 0.0)
    out_ref[0] = Fv2
    mean_ref[0, 0] = jnp.mean(Fv2, axis=0)


@functools.partial(jax.jit, static_argnames=())
def kernel(loc, deadline, depot, W0_w, W0_b, W2_w, W2_b):
    B = loc.shape[0]
    locations = jnp.concatenate([depot[:, None, :], loc], axis=1)
    td = jnp.concatenate(
        [jnp.zeros((B, 1), deadline.dtype), deadline], axis=1)
    feat = jnp.concatenate([locations, td[..., None]], axis=-1)
    featT = jnp.swapaxes(feat, 1, 2)
    w0t = W0_w.T
    w2t = W2_w.T
    w0b = W0_b[None, :]
    w2b = W2_b[None, :]

    grid = (B,)
    out_shape = (
        jax.ShapeDtypeStruct((B, _N, _E), jnp.float32),
        jax.ShapeDtypeStruct((B, 1, _E), jnp.float32),
    )
    Fv2, mean = pl.pallas_call(
        _ccn2_body,
        grid=grid,
        in_specs=[
            pl.BlockSpec((1, _N, 3), lambda b: (b, 0, 0)),
            pl.BlockSpec((1, 3, _N), lambda b: (b, 0, 0)),
            pl.BlockSpec((3, _E), lambda b: (0, 0)),
            pl.BlockSpec((1, _E), lambda b: (0, 0)),
            pl.BlockSpec((_E, _E), lambda b: (0, 0)),
            pl.BlockSpec((1, _E), lambda b: (0, 0)),
        ],
        out_specs=(
            pl.BlockSpec((1, _N, _E), lambda b: (b, 0, 0)),
            pl.BlockSpec((1, 1, _E), lambda b: (b, 0, 0)),
        ),
        out_shape=out_shape,
        compiler_params=pltpu.CompilerParams(
            dimension_semantics=("arbitrary",),
        ),
    )(feat, featT, w0t, w0b, w2t, w2b)
    return Fv2, mean[:, 0, :]


# speedup vs baseline: 1.2118x; 1.1891x over previous
import functools

import jax
import jax.numpy as jnp
from jax.experimental import pallas as pl
from jax.experimental.pallas import tpu as pltpu

_THRESH = 0.055
_N = 500
_E = 128
_S = 4


def _one_sample(f, ft, w0t, w0b, w2t, w2b):
    xc = f[:, 0:1]
    yc = f[:, 1:2]
    xr = ft[0:1, :]
    yr = ft[1:2, :]
    dx = xc - xr
    dy = yc - yr
    dist2 = dx * dx + dy * dy
    A = (dist2 <= _THRESH * _THRESH).astype(jnp.float32)
    Ab = A.astype(jnp.bfloat16)

    fv0 = jnp.maximum(
        jnp.dot(f, w0t, preferred_element_type=jnp.float32) + w0b, 0.0)
    fv1 = jnp.dot(A, fv0, preferred_element_type=jnp.float32)

    C = jnp.dot(Ab, Ab, preferred_element_type=jnp.float32)
    B2 = (C > 0).astype(jnp.float32)
    D = jnp.dot(B2.astype(jnp.bfloat16), Ab, preferred_element_type=jnp.float32)
    M = B2 * D

    fv2 = jnp.dot(M, fv1, preferred_element_type=jnp.float32)
    Fv2 = jnp.maximum(
        jnp.dot(fv2, w2t, preferred_element_type=jnp.float32) + w2b, 0.0)
    return Fv2


def _ccn2_body(feat_ref, featT_ref, w0t_ref, w0b_ref, w2t_ref, w2b_ref,
               out_ref, mean_ref):
    for s in range(_S):
        Fv2 = _one_sample(feat_ref[s], featT_ref[s], w0t_ref[...],
                          w0b_ref[...], w2t_ref[...], w2b_ref[...])
        out_ref[s] = Fv2
        mean_ref[s, 0] = jnp.mean(Fv2, axis=0)


@functools.partial(jax.jit, static_argnames=())
def kernel(loc, deadline, depot, W0_w, W0_b, W2_w, W2_b):
    B = loc.shape[0]
    locations = jnp.concatenate([depot[:, None, :], loc], axis=1)
    td = jnp.concatenate(
        [jnp.zeros((B, 1), deadline.dtype), deadline], axis=1)
    feat = jnp.concatenate([locations, td[..., None]], axis=-1)
    featT = jnp.swapaxes(feat, 1, 2)
    w0t = W0_w.T
    w2t = W2_w.T
    w0b = W0_b[None, :]
    w2b = W2_b[None, :]

    grid = (B // _S,)
    out_shape = (
        jax.ShapeDtypeStruct((B, _N, _E), jnp.float32),
        jax.ShapeDtypeStruct((B, 1, _E), jnp.float32),
    )
    Fv2, mean = pl.pallas_call(
        _ccn2_body,
        grid=grid,
        in_specs=[
            pl.BlockSpec((_S, _N, 3), lambda b: (b, 0, 0)),
            pl.BlockSpec((_S, 3, _N), lambda b: (b, 0, 0)),
            pl.BlockSpec((3, _E), lambda b: (0, 0)),
            pl.BlockSpec((1, _E), lambda b: (0, 0)),
            pl.BlockSpec((_E, _E), lambda b: (0, 0)),
            pl.BlockSpec((1, _E), lambda b: (0, 0)),
        ],
        out_specs=(
            pl.BlockSpec((_S, _N, _E), lambda b: (b, 0, 0)),
            pl.BlockSpec((_S, 1, _E), lambda b: (b, 0, 0)),
        ),
        out_shape=out_shape,
        compiler_params=pltpu.CompilerParams(
            dimension_semantics=("arbitrary",),
        ),
    )(feat, featT, w0t, w0b, w2t, w2b)
    return Fv2, mean[:, 0, :]
